# baseline (speedup 1.0000x reference)
import functools

import jax
import jax.numpy as jnp
from jax.experimental import pallas as pl

_D = 128
_NH = 32
_R = 2048


def _search16(y, m_ref, t_ref):
    def m(k):
        return m_ref[k : k + 1, :]

    def t(k):
        return t_ref[k : k + 1, :]

    w = jnp.where
    b3 = y > m(8)
    b2 = y > w(b3, m(12), m(4))
    b1 = y > w(b3, w(b2, m(14), m(10)), w(b2, m(6), m(2)))
    b0 = y > w(
        b3,
        w(b2, w(b1, m(15), m(13)), w(b1, m(11), m(9))),
        w(b2, w(b1, m(7), m(5)), w(b1, m(3), m(1))),
    )
    return w(
        b3,
        w(
            b2,
            w(b1, w(b0, t(15), t(14)), w(b0, t(13), t(12))),
            w(b1, w(b0, t(11), t(10)), w(b0, t(9), t(8))),
        ),
        w(
            b2,
            w(b1, w(b0, t(7), t(6)), w(b0, t(5), t(4))),
            w(b1, w(b0, t(3), t(2)), w(b0, t(1), t(0))),
        ),
    )


def _search4(y, m_ref, t_ref):
    def m(k):
        return m_ref[k : k + 1, :]

    def t(k):
        return t_ref[k : k + 1, :]

    w = jnp.where
    b1 = y > m(2)
    b0 = y > w(b1, m(3), m(1))
    return w(b1, w(b0, t(3), t(2)), w(b0, t(1), t(0)))


def _body(x_ref, pipt_ref, s_ref, a_ref, plo_ref, hm_ref, ht_ref, lm_ref, lt_ref, o_ref):
    f32 = jnp.float32
    ys = []
    yh = None
    for j in range(4):
        y_j = jnp.dot(
            x_ref[:, 128 * j : 128 * (j + 1)], pipt_ref[...], preferred_element_type=f32
        )
        ys.append(y_j)
        p = jnp.dot(y_j, s_ref[128 * j : 128 * (j + 1), :], preferred_element_type=f32)
        yh = p if yh is None else yh + p
    yqh = _search16(yh, hm_ref, ht_ref)
    for j in range(4):
        yql_j = _search4(ys[j], lm_ref, lt_ref)
        o_ref[:, 128 * j : 128 * (j + 1)] = jnp.dot(
            yql_j, plo_ref[...], preferred_element_type=f32
        ) + jnp.dot(yqh, a_ref[128 * j : 128 * (j + 1), :], preferred_element_type=f32)


def _mid_rows(tbl, k):
    mids = jnp.concatenate([jnp.zeros((1,), jnp.float32), 0.5 * (tbl[1:] + tbl[:-1])])
    return jnp.broadcast_to(mids[:, None], (k, _D))


@functools.partial(jax.jit, static_argnames=())
def kernel(x, Pi, high_centroids, low_centroids, high_indices, low_indices):
    B = x.shape[0]
    perm = jnp.concatenate([high_indices, low_indices]).astype(jnp.int32)
    PiP = Pi[perm, :]
    PiPT = PiP.T
    plo = PiP * (jnp.arange(_D)[:, None] >= _NH).astype(jnp.float32)
    jj = jnp.repeat(jnp.arange(4), _NH)
    cc = jnp.tile(jnp.arange(_NH), 4)
    S = (
        jnp.zeros((4, _D, _D), jnp.float32)
        .at[jj, cc, jnp.arange(_D)]
        .set(1.0)
        .reshape(4 * _D, _D)
    )
    A = (
        jnp.zeros((4, _D, _D), jnp.float32)
        .at[jj, jnp.arange(_D), :]
        .set(PiP[cc, :])
        .reshape(4 * _D, _D)
    )
    hm = _mid_rows(high_centroids, 16)
    ht = jnp.broadcast_to(high_centroids[:, None], (16, _D))
    lm = _mid_rows(low_centroids, 4)
    lt = jnp.broadcast_to(low_centroids[:, None], (4, _D))

    x4 = x.reshape(B // 4, 4 * _D)
    r = min(_R, B // 4)
    grid = ((B // 4) // r,)
    out = pl.pallas_call(
        _body,
        grid=grid,
        in_specs=[
            pl.BlockSpec((r, 4 * _D), lambda i: (i, 0)),
            pl.BlockSpec((_D, _D), lambda i: (0, 0)),
            pl.BlockSpec((4 * _D, _D), lambda i: (0, 0)),
            pl.BlockSpec((4 * _D, _D), lambda i: (0, 0)),
            pl.BlockSpec((_D, _D), lambda i: (0, 0)),
            pl.BlockSpec((16, _D), lambda i: (0, 0)),
            pl.BlockSpec((16, _D), lambda i: (0, 0)),
            pl.BlockSpec((4, _D), lambda i: (0, 0)),
            pl.BlockSpec((4, _D), lambda i: (0, 0)),
        ],
        out_specs=pl.BlockSpec((r, 4 * _D), lambda i: (i, 0)),
        out_shape=jax.ShapeDtypeStruct((B // 4, 4 * _D), jnp.float32),
    )(x4, PiPT, S, A, plo, hm, ht, lm, lt)
    return out.reshape(B, _D)


# speedup vs baseline: 3.0550x; 3.0550x over previous
import functools

import jax
import jax.numpy as jnp
from jax.experimental import pallas as pl

_D = 128
_NH = 32
_BLK = 8192


def _search16(y, m_ref, t_ref):
    def m(k):
        return m_ref[k : k + 1, :]

    def t(k):
        return t_ref[k : k + 1, :]

    w = jnp.where
    b3 = y > m(8)
    b2 = y > w(b3, m(12), m(4))
    b1 = y > w(b3, w(b2, m(14), m(10)), w(b2, m(6), m(2)))
    b0 = y > w(
        b3,
        w(b2, w(b1, m(15), m(13)), w(b1, m(11), m(9))),
        w(b2, w(b1, m(7), m(5)), w(b1, m(3), m(1))),
    )
    return w(
        b3,
        w(
            b2,
            w(b1, w(b0, t(15), t(14)), w(b0, t(13), t(12))),
            w(b1, w(b0, t(11), t(10)), w(b0, t(9), t(8))),
        ),
        w(
            b2,
            w(b1, w(b0, t(7), t(6)), w(b0, t(5), t(4))),
            w(b1, w(b0, t(3), t(2)), w(b0, t(1), t(0))),
        ),
    )


def _search4(y, m_ref, t_ref):
    def m(k):
        return m_ref[k : k + 1, :]

    def t(k):
        return t_ref[k : k + 1, :]

    w = jnp.where
    b1 = y > m(2)
    b0 = y > w(b1, m(3), m(1))
    return w(b1, w(b0, t(3), t(2)), w(b0, t(1), t(0)))


def _body(x_ref, pipt_ref, pip_ref, hm_ref, ht_ref, lm_ref, lt_ref, o_ref):
    f32 = jnp.float32
    blk = x_ref.shape[0]
    y = jnp.dot(x_ref[...], pipt_ref[...], preferred_element_type=f32)
    q = blk // 4
    yhp = jnp.concatenate(
        [y[j * q : (j + 1) * q, :_NH] for j in range(4)], axis=1
    )
    yqh4 = _search16(yhp, hm_ref, ht_ref)
    yqh = jnp.concatenate(
        [yqh4[:, j * _NH : (j + 1) * _NH] for j in range(4)], axis=0
    )
    yql = _search4(y, lm_ref, lt_ref)
    yq = jnp.concatenate([yqh, yql[:, _NH:]], axis=1)
    o_ref[...] = jnp.dot(yq, pip_ref[...], preferred_element_type=f32)


def _mid_rows(tbl, k):
    mids = jnp.concatenate([jnp.zeros((1,), jnp.float32), 0.5 * (tbl[1:] + tbl[:-1])])
    return jnp.broadcast_to(mids[:, None], (k, _D))


@functools.partial(jax.jit, static_argnames=())
def kernel(x, Pi, high_centroids, low_centroids, high_indices, low_indices):
    B = x.shape[0]
    perm = jnp.concatenate([high_indices, low_indices]).astype(jnp.int32)
    PiP = Pi[perm, :]
    hm = _mid_rows(high_centroids, 16)
    ht = jnp.broadcast_to(high_centroids[:, None], (16, _D))
    lm = _mid_rows(low_centroids, 4)
    lt = jnp.broadcast_to(low_centroids[:, None], (4, _D))

    blk = min(_BLK, B)
    grid = (B // blk,)
    return pl.pallas_call(
        _body,
        grid=grid,
        in_specs=[
            pl.BlockSpec((blk, _D), lambda i: (i, 0)),
            pl.BlockSpec((_D, _D), lambda i: (0, 0)),
            pl.BlockSpec((_D, _D), lambda i: (0, 0)),
            pl.BlockSpec((16, _D), lambda i: (0, 0)),
            pl.BlockSpec((16, _D), lambda i: (0, 0)),
            pl.BlockSpec((4, _D), lambda i: (0, 0)),
            pl.BlockSpec((4, _D), lambda i: (0, 0)),
        ],
        out_specs=pl.BlockSpec((blk, _D), lambda i: (i, 0)),
        out_shape=jax.ShapeDtypeStruct((B, _D), jnp.float32),
    )(x, PiP.T, PiP, hm, ht, lm, lt)


# speedup vs baseline: 3.5979x; 1.1777x over previous
import functools

import jax
import jax.numpy as jnp
from jax.experimental import pallas as pl
from jax.experimental.pallas import tpu as pltpu

_D = 128
_NH = 32
_BLK = 8192


def _row(s):
    return jnp.full((1, _D), s, jnp.float32)


def _search16(y, hs_ref):
    hs = [hs_ref[k] for k in range(16)]
    m = [None] + [0.5 * (hs[k - 1] + hs[k]) for k in range(1, 16)]
    w = jnp.where
    b3 = y > m[8]
    b2 = y > w(b3, m[12], m[4])
    b1 = y > w(b3, w(b2, m[14], m[10]), w(b2, m[6], m[2]))
    b0 = y > w(
        b3,
        w(b2, w(b1, m[15], m[13]), w(b1, m[11], m[9])),
        w(b2, w(b1, m[7], m[5]), w(b1, m[3], m[1])),
    )
    idx3 = (
        b2.astype(jnp.int32) * 4 + b1.astype(jnp.int32) * 2 + b0.astype(jnp.int32)
    )
    tlo = jnp.concatenate([_row(hs[k]) for k in range(8)], axis=0)
    thi = jnp.concatenate([_row(hs[k]) for k in range(8, 16)], axis=0)
    glo = jnp.take_along_axis(tlo, idx3, axis=0)
    ghi = jnp.take_along_axis(thi, idx3, axis=0)
    return w(b3, ghi, glo)


def _search4(y, ls_ref):
    ls = [ls_ref[k] for k in range(4)]
    m = [None] + [0.5 * (ls[k - 1] + ls[k]) for k in range(1, 4)]
    w = jnp.where
    b1 = y > m[2]
    b0 = y > w(b1, m[3], m[1])
    return w(b1, w(b0, ls[3], ls[2]), w(b0, ls[1], ls[0]))


def _body(x_ref, p_ref, pi_ref, hs_ref, ls_ref, o_ref):
    f32 = jnp.float32
    blk = x_ref.shape[0]
    pip = jnp.dot(p_ref[...], pi_ref[...], preferred_element_type=f32)
    y = jax.lax.dot_general(
        x_ref[...], pip, (((1,), (1,)), ((), ())), preferred_element_type=f32
    )
    q = blk // 4
    yhp = jnp.concatenate(
        [y[j * q : (j + 1) * q, :_NH] for j in range(4)], axis=1
    )
    yqh4 = _search16(yhp, hs_ref)
    yqh = jnp.concatenate(
        [yqh4[:, j * _NH : (j + 1) * _NH] for j in range(4)], axis=0
    )
    yql = _search4(y, ls_ref)
    yq = jnp.concatenate([yqh, yql[:, _NH:]], axis=1)
    o_ref[...] = jnp.dot(yq, pip, preferred_element_type=f32)


@functools.partial(jax.jit, static_argnames=())
def kernel(x, Pi, high_centroids, low_centroids, high_indices, low_indices):
    B = x.shape[0]
    perm = jnp.concatenate([high_indices, low_indices]).astype(jnp.int32)
    P = jax.nn.one_hot(perm, _D, dtype=jnp.float32)

    blk = min(_BLK, B)
    grid = (B // blk,)
    return pl.pallas_call(
        _body,
        grid=grid,
        in_specs=[
            pl.BlockSpec((blk, _D), lambda i: (i, 0)),
            pl.BlockSpec((_D, _D), lambda i: (0, 0)),
            pl.BlockSpec((_D, _D), lambda i: (0, 0)),
            pl.BlockSpec(memory_space=pltpu.SMEM),
            pl.BlockSpec(memory_space=pltpu.SMEM),
        ],
        out_specs=pl.BlockSpec((blk, _D), lambda i: (i, 0)),
        out_shape=jax.ShapeDtypeStruct((B, _D), jnp.float32),
        compiler_params=pltpu.CompilerParams(dimension_semantics=("parallel",)),
    )(x, P, Pi, high_centroids, low_centroids)


# speedup vs baseline: 3.6250x; 1.0075x over previous
import functools

import jax
import jax.numpy as jnp
from jax.experimental import pallas as pl
from jax.experimental.pallas import tpu as pltpu

_D = 128
_NH = 32
_BLK = 8192


def _row(s):
    return jnp.full((1, _D), s, jnp.float32)


def _search16(y, hs_ref):
    hs = [hs_ref[k] for k in range(16)]
    m = [None] + [0.5 * (hs[k - 1] + hs[k]) for k in range(1, 16)]
    w = jnp.where
    b3 = y > m[8]
    b2 = y > w(b3, m[12], m[4])
    b1 = y > w(b3, w(b2, m[14], m[10]), w(b2, m[6], m[2]))
    b0 = y > w(
        b3,
        w(b2, w(b1, m[15], m[13]), w(b1, m[11], m[9])),
        w(b2, w(b1, m[7], m[5]), w(b1, m[3], m[1])),
    )
    i4 = jnp.where(b2, jnp.int32(4), jnp.int32(0))
    i2 = jnp.where(b1, jnp.int32(2), jnp.int32(0))
    i1 = jnp.where(b0, jnp.int32(1), jnp.int32(0))
    idx3 = i4 + i2 + i1
    tlo = jnp.concatenate([_row(hs[k]) for k in range(8)], axis=0)
    thi = jnp.concatenate([_row(hs[k]) for k in range(8, 16)], axis=0)
    glo = jnp.take_along_axis(tlo, idx3, axis=0)
    ghi = jnp.take_along_axis(thi, idx3, axis=0)
    return w(b3, ghi, glo)


def _search4(y, ls_ref):
    ls = [ls_ref[k] for k in range(4)]
    m = [None] + [0.5 * (ls[k - 1] + ls[k]) for k in range(1, 4)]
    w = jnp.where
    b1 = y > m[2]
    b0 = y > w(b1, m[3], m[1])
    return w(b1, w(b0, ls[3], ls[2]), w(b0, ls[1], ls[0]))


def _body(x_ref, p_ref, pi_ref, hs_ref, ls_ref, o_ref):
    f32 = jnp.float32
    blk = x_ref.shape[0]
    pip = jnp.dot(p_ref[...], pi_ref[...], preferred_element_type=f32)
    y = jax.lax.dot_general(
        x_ref[...], pip, (((1,), (1,)), ((), ())), preferred_element_type=f32
    )
    q = blk // 4
    yhp = jnp.concatenate(
        [y[j * q : (j + 1) * q, :_NH] for j in range(4)], axis=1
    )
    yqh4 = _search16(yhp, hs_ref)
    yqh = jnp.concatenate(
        [yqh4[:, j * _NH : (j + 1) * _NH] for j in range(4)], axis=0
    )
    yql = _search4(y, ls_ref)
    yq = jnp.concatenate([yqh, yql[:, _NH:]], axis=1)
    o_ref[...] = jnp.dot(yq, pip, preferred_element_type=f32)


@functools.partial(jax.jit, static_argnames=())
def kernel(x, Pi, high_centroids, low_centroids, high_indices, low_indices):
    B = x.shape[0]
    perm = jnp.concatenate([high_indices, low_indices]).astype(jnp.int32)
    P = jax.nn.one_hot(perm, _D, dtype=jnp.float32)

    blk = min(_BLK, B)
    grid = (B // blk,)
    return pl.pallas_call(
        _body,
        grid=grid,
        in_specs=[
            pl.BlockSpec((blk, _D), lambda i: (i, 0)),
            pl.BlockSpec((_D, _D), lambda i: (0, 0)),
            pl.BlockSpec((_D, _D), lambda i: (0, 0)),
            pl.BlockSpec(memory_space=pltpu.SMEM),
            pl.BlockSpec(memory_space=pltpu.SMEM),
        ],
        out_specs=pl.BlockSpec((blk, _D), lambda i: (i, 0)),
        out_shape=jax.ShapeDtypeStruct((B, _D), jnp.float32),
        compiler_params=pltpu.CompilerParams(dimension_semantics=("parallel",)),
    )(x, P, Pi, high_centroids, low_centroids)
